# baseline (speedup 1.0000x reference)
import functools

import jax
import jax.numpy as jnp
import numpy as np
from jax import lax
from jax.experimental import pallas as pl
from jax.experimental.pallas import tpu as pltpu
from jax.experimental.pallas import tpu_sc as plsc

_G = 2
_R = 2
_CDIM = 4
_DPG = 512
_HALF_L = 4.0 * (1.0 + 1e-3) / 2.0
_HALF_W = 2.0
_BASIS = (1.0, 5.0, 25.0, 125.0)

_B_TC = 3
_CTS = 128
_NW = 32


def _fsq_tc_kernel(w_ref, b_ref, basis_ref, x_ref, o_ref):
    xb = x_ref[0]
    w = w_ref[...]
    b = b_ref[...]
    z = jax.lax.dot_general(
        w.astype(jnp.bfloat16), xb.astype(jnp.bfloat16), (((1,), (0,)), ((), ())),
        preferred_element_type=jnp.float32,
    ) + b
    r0 = jnp.round(jnp.tanh(z) * _HALF_L)
    resid = z - r0 * (1.0 / _HALF_W)
    r1 = jnp.round(jnp.tanh(resid * 4.0) * _HALF_L)
    basis8 = basis_ref[...]
    w0 = (r0 + _HALF_W) * basis8
    w1 = (r1 + _HALF_W) * basis8
    row = [
        jnp.sum(w0[0:4], axis=0, keepdims=True),
        jnp.sum(w1[0:4], axis=0, keepdims=True),
        jnp.sum(w0[4:8], axis=0, keepdims=True),
        jnp.sum(w1[4:8], axis=0, keepdims=True),
    ]
    o_ref[0] = jnp.concatenate(row, axis=0).astype(jnp.int32)


def _tc_part(x, w8, b8, basis8):
    _, D, T = x.shape
    return pl.pallas_call(
        _fsq_tc_kernel,
        grid=(_B_TC,),
        in_specs=[
            pl.BlockSpec((_G * _CDIM, D), lambda bi: (0, 0)),
            pl.BlockSpec((_G * _CDIM, 1), lambda bi: (0, 0)),
            pl.BlockSpec((_G * _CDIM, 1), lambda bi: (0, 0)),
            pl.BlockSpec((1, D, T), lambda bi: (bi, 0, 0)),
        ],
        out_specs=pl.BlockSpec((1, _G * _R, T), lambda bi: (bi, 0, 0)),
        out_shape=jax.ShapeDtypeStruct((_B_TC, _G * _R, T), jnp.int32),
        compiler_params=pltpu.CompilerParams(
            dimension_semantics=("parallel",),
        ),
    )(w8, b8, basis8, x)


def _bf16_rtne(v):
    u = lax.bitcast_convert_type(v, jnp.int32)
    r = lax.shift_right_logical(u, 16) & 1
    u = u + (0x7FFF + r)
    u = u & jnp.int32(-65536)
    return lax.bitcast_convert_type(u, jnp.float32)


def _round_rtne(v):
    c = jnp.float32(1.5 * 2.0 ** 23)
    return (v + c) - c


def _tanh_sc(v):
    e = jnp.exp(jnp.minimum(2.0 * v, 60.0))
    return (e - 1.0) / (e + 1.0)


def _sc_body(x_hbm, w_hbm, b_hbm, out_hbm, x_v, w_v, b_v, o_v):
    wid = lax.axis_index("s") * 2 + lax.axis_index("c")
    g = wid // 16
    col0 = (wid % 16) * _CTS
    pltpu.sync_copy(w_hbm.at[g], w_v)
    pltpu.sync_copy(b_hbm.at[g], b_v)

    nstrip = _CTS // 16
    dh = _DPG // 2
    zero = jnp.zeros((16,), jnp.float32)

    accs = tuple([zero] * (nstrip * _CDIM))
    for h in range(_DPG // dh):

        def dstep(d, accs, h=h):
            new = list(accs)
            wv = w_v[h * dh + d]
            ws = [wv[c] for c in range(_CDIM)]
            for s in range(nstrip):
                xs = _bf16_rtne(x_v[d, pl.ds(16 * s, 16)])
                for c in range(_CDIM):
                    new[s * _CDIM + c] = new[s * _CDIM + c] + ws[c] * xs
            return tuple(new)

        pltpu.sync_copy(
            x_hbm.at[_B_TC, pl.ds(g * _DPG + h * dh, dh), pl.ds(col0, _CTS)], x_v
        )
        accs = lax.fori_loop(0, dh, dstep, accs)

    bv = b_v[...]
    for s in range(nstrip):
        idx0 = zero
        idx1 = zero
        for c in range(_CDIM):
            z = accs[s * _CDIM + c] + bv[c]
            r0 = _round_rtne(_tanh_sc(z) * _HALF_L)
            resid = z - r0 * (1.0 / _HALF_W)
            r1 = _round_rtne(_tanh_sc(resid * 4.0) * _HALF_L)
            idx0 = idx0 + (r0 + _HALF_W) * _BASIS[c]
            idx1 = idx1 + (r1 + _HALF_W) * _BASIS[c]
        o_v[0, pl.ds(16 * s, 16)] = idx0.astype(jnp.int32)
        o_v[1, pl.ds(16 * s, 16)] = idx1.astype(jnp.int32)

    pltpu.sync_copy(o_v, out_hbm.at[g, :, pl.ds(col0, _CTS)])


def _sc_part(x, wk, b16):
    T = x.shape[2]
    mesh = plsc.VectorSubcoreMesh(core_axis_name="c", subcore_axis_name="s")
    f = functools.partial(
        pl.kernel,
        out_type=jax.ShapeDtypeStruct((_G, _R, T), jnp.int32),
        mesh=mesh,
        scratch_types=[
            pltpu.VMEM((_DPG // 2, _CTS), jnp.float32),
            pltpu.VMEM((_DPG, 16), jnp.float32),
            pltpu.VMEM((16,), jnp.float32),
            pltpu.VMEM((_R, _CTS), jnp.int32),
        ],
    )(_sc_body)
    return f(x, wk, b16)


def kernel(x, Win, b_in, Wout, b_out):
    del Wout, b_out
    B, D, T = x.shape
    w8 = jnp.zeros((_G * _CDIM, D), dtype=jnp.float32)
    w8 = w8.at[0:4, 0:512].set(Win[0]).at[4:8, 512:1024].set(Win[1])
    b8 = jnp.concatenate([b_in[0], b_in[1]]).reshape(_G * _CDIM, 1)
    basis8 = jnp.asarray(_BASIS * _G, dtype=jnp.float32).reshape(_G * _CDIM, 1)
    wk = Win.astype(jnp.bfloat16).astype(jnp.float32)
    wk = jnp.concatenate(
        [jnp.transpose(wk, (0, 2, 1)), jnp.zeros((_G, _DPG, 12), jnp.float32)],
        axis=2,
    )
    b16 = jnp.concatenate([b_in, jnp.zeros((_G, 12), jnp.float32)], axis=1)

    out_tc = _tc_part(x, w8, b8, basis8)
    out_sc = _sc_part(x, wk, b16).reshape(1, _G * _R, T)
    return jnp.concatenate([out_tc, out_sc], axis=0)


# speedup vs baseline: 2.7315x; 2.7315x over previous
import jax
import jax.numpy as jnp
import numpy as np
from jax.experimental import pallas as pl
from jax.experimental.pallas import tpu as pltpu

_G = 2
_R = 2
_CDIM = 4
_DPG = 512
_HALF_L = 4.0 * (1.0 + 1e-3) / 2.0
_HALF_W = 2.0
_BASIS = (1.0, 5.0, 25.0, 125.0)
_TT = 2048
_CT = 512
_LANES = 128


def _fsq_rows(z, basis4):
    r0 = jnp.round(jnp.tanh(z) * _HALF_L)
    resid = z - r0 * (1.0 / _HALF_W)
    r1 = jnp.round(jnp.tanh(resid * 4.0) * _HALF_L)
    i0 = jnp.sum((r0 + _HALF_W) * basis4, axis=0, keepdims=True)
    i1 = jnp.sum((r1 + _HALF_W) * basis4, axis=0, keepdims=True)
    return i0, i1


def _fsq_kernel(w0_ref, wrep_ref, b_ref, basis_ref, x_ref, o_ref):
    basis4 = basis_ref[0:4]
    x0 = x_ref[0, 0:_DPG, :]
    z0 = jax.lax.dot_general(
        w0_ref[...].astype(jnp.bfloat16), x0.astype(jnp.bfloat16),
        (((1,), (0,)), ((), ())),
        preferred_element_type=jnp.float32,
    ) + b_ref[0:4]
    i00, i01 = _fsq_rows(z0, basis4)
    o_ref[0, 0:1, :] = i00.astype(jnp.int32)
    o_ref[0, 1:2, :] = i01.astype(jnp.int32)
    for kk in range(_TT // _CT):
        cs = slice(kk * _CT, (kk + 1) * _CT)
        accs = [None] * _CDIM
        for j in range(_DPG // 8):
            xs = x_ref[0, _DPG + 8 * j:_DPG + 8 * (j + 1), cs]
            xs = xs.astype(jnp.bfloat16).astype(jnp.float32)
            for c in range(_CDIM):
                wv = wrep_ref[c, 8 * j:8 * (j + 1), :]
                wt = jnp.tile(wv, (1, _CT // _LANES))
                p = wt * xs
                accs[c] = p if accs[c] is None else accs[c] + p
        z1 = jnp.concatenate(
            [jnp.sum(a, axis=0, keepdims=True) for a in accs], axis=0
        ) + b_ref[4:8]
        i10, i11 = _fsq_rows(z1, basis4)
        o_ref[0, 2:3, cs] = i10.astype(jnp.int32)
        o_ref[0, 3:4, cs] = i11.astype(jnp.int32)


def kernel(x, Win, b_in, Wout, b_out):
    del Wout, b_out
    B, D, T = x.shape
    w0 = Win[0]
    w1 = Win[1].astype(jnp.bfloat16).astype(jnp.float32)
    wrep = jnp.broadcast_to(w1[:, :, None], (_CDIM, _DPG, _LANES))
    b8 = jnp.concatenate([b_in[0], b_in[1]]).reshape(_G * _CDIM, 1)
    basis8 = jnp.asarray(_BASIS * _G, dtype=jnp.float32).reshape(_G * _CDIM, 1)
    grid = (B, T // _TT)
    out = pl.pallas_call(
        _fsq_kernel,
        grid=grid,
        in_specs=[
            pl.BlockSpec((_CDIM, _DPG), lambda bi, ti: (0, 0)),
            pl.BlockSpec((_CDIM, _DPG, _LANES), lambda bi, ti: (0, 0, 0)),
            pl.BlockSpec((_G * _CDIM, 1), lambda bi, ti: (0, 0)),
            pl.BlockSpec((_G * _CDIM, 1), lambda bi, ti: (0, 0)),
            pl.BlockSpec((1, D, _TT), lambda bi, ti: (bi, 0, ti)),
        ],
        out_specs=pl.BlockSpec((1, _G * _R, _TT), lambda bi, ti: (bi, 0, ti)),
        out_shape=jax.ShapeDtypeStruct((B, _G * _R, T), jnp.int32),
        compiler_params=pltpu.CompilerParams(
            dimension_semantics=("parallel", "parallel"),
        ),
    )(w0, wrep, b8, basis8, x)
    return out
